# baseline (speedup 1.0000x reference)
import functools

import jax
import jax.numpy as jnp
from jax import lax
from jax.experimental import pallas as pl
from jax.experimental.pallas import tpu as pltpu
from jax.experimental.pallas import tpu_sc as plsc

NUM_NODES = 100000
EMBED_DIM = 64
BATCH = 4096
LANES = 16


def _build():
    info = plsc.get_sparse_core_info()
    num_cores, num_subcores = info.num_cores, info.num_subcores
    num_workers = num_cores * num_subcores
    rows_per_w = EMBED_DIM // num_workers
    mesh = plsc.VectorSubcoreMesh(core_axis_name="c", subcore_axis_name="s")

    @functools.partial(
        pl.kernel,
        mesh=mesh,
        out_type=jax.ShapeDtypeStruct((EMBED_DIM, BATCH), jnp.float32),
        compiler_params=pltpu.CompilerParams(needs_layout_passes=False),
        scratch_types=[
            pltpu.VMEM((BATCH,), jnp.int32),
            pltpu.VMEM((NUM_NODES,), jnp.float32),
            pltpu.VMEM((BATCH,), jnp.float32),
            pltpu.VMEM((BATCH,), jnp.float32),
            pltpu.SemaphoreType.DMA,
            pltpu.SemaphoreType.DMA,
        ],
    )
    def gather_kernel(idx_hbm, tab_t_hbm, out_t_hbm, idx_v, row_v,
                      out0_v, out1_v, sem_r, sem_w):
        wid = lax.axis_index("s") * num_cores + lax.axis_index("c")
        j0 = wid * rows_per_w
        j1 = j0 + 1

        s0 = pltpu.async_copy(tab_t_hbm.at[j0], row_v, sem_r)
        pltpu.sync_copy(idx_hbm, idx_v)

        def gather_row(out_v):
            @plsc.parallel_loop(0, BATCH, step=LANES, unroll=8)
            def _(i):
                idxv = idx_v[pl.ds(i, LANES)]
                out_v[pl.ds(i, LANES)] = plsc.load_gather(row_v, [idxv])

        s0.wait()
        gather_row(out0_v)
        s1 = pltpu.async_copy(tab_t_hbm.at[j1], row_v, sem_r)
        w0 = pltpu.async_copy(out0_v, out_t_hbm.at[j0], sem_w)
        s1.wait()
        gather_row(out1_v)
        w0.wait()
        pltpu.sync_copy(out1_v, out_t_hbm.at[j1])

    return gather_kernel


_gather = _build()


def kernel(node_id, table):
    out_t = _gather(node_id.astype(jnp.int32), table.T)
    return out_t.T


# speedup vs baseline: 1.0042x; 1.0042x over previous
import functools

import jax
import jax.numpy as jnp
from jax import lax
from jax.experimental import pallas as pl
from jax.experimental.pallas import tpu as pltpu
from jax.experimental.pallas import tpu_sc as plsc

NUM_NODES = 100000
EMBED_DIM = 64
BATCH = 4096
LANES = 16


def _build():
    info = plsc.get_sparse_core_info()
    num_cores, num_subcores = info.num_cores, info.num_subcores
    num_workers = num_cores * num_subcores
    rows_per_w = EMBED_DIM // num_workers
    mesh = plsc.VectorSubcoreMesh(core_axis_name="c", subcore_axis_name="s")

    @functools.partial(
        pl.kernel,
        mesh=mesh,
        out_type=jax.ShapeDtypeStruct((EMBED_DIM, BATCH), jnp.float32),
        compiler_params=pltpu.CompilerParams(
            needs_layout_passes=False,
            disable_bounds_checks=True,
            disable_semaphore_checks=True,
        ),
        scratch_types=[
            pltpu.VMEM((BATCH,), jnp.int32),
            pltpu.VMEM((NUM_NODES,), jnp.float32),
            pltpu.VMEM((BATCH,), jnp.float32),
            pltpu.VMEM((BATCH,), jnp.float32),
            pltpu.SemaphoreType.DMA,
            pltpu.SemaphoreType.DMA,
        ],
    )
    def gather_kernel(idx_hbm, tab_t_hbm, out_t_hbm, idx_v, row_v,
                      out0_v, out1_v, sem_r, sem_w):
        wid = lax.axis_index("s") * num_cores + lax.axis_index("c")
        j0 = wid * rows_per_w
        j1 = j0 + 1

        s0 = pltpu.async_copy(tab_t_hbm.at[j0], row_v, sem_r)
        pltpu.sync_copy(idx_hbm, idx_v)

        def gather_row(out_v):
            @plsc.parallel_loop(0, BATCH, step=LANES, unroll=8)
            def _(i):
                idxv = idx_v[pl.ds(i, LANES)]
                out_v[pl.ds(i, LANES)] = plsc.load_gather(row_v, [idxv])

        s0.wait()
        gather_row(out0_v)
        s1 = pltpu.async_copy(tab_t_hbm.at[j1], row_v, sem_r)
        w0 = pltpu.async_copy(out0_v, out_t_hbm.at[j0], sem_w)
        s1.wait()
        gather_row(out1_v)
        w0.wait()
        pltpu.sync_copy(out1_v, out_t_hbm.at[j1])

    return gather_kernel


_gather = _build()


def kernel(node_id, table):
    out_t = _gather(node_id.astype(jnp.int32), table.T)
    return out_t.T


# speedup vs baseline: 1.0043x; 1.0001x over previous
import functools

import jax
import jax.numpy as jnp
from jax import lax
from jax.experimental import pallas as pl
from jax.experimental.pallas import tpu as pltpu
from jax.experimental.pallas import tpu_sc as plsc

NUM_NODES = 100000
EMBED_DIM = 64
BATCH = 4096
LANES = 16


def _build():
    info = plsc.get_sparse_core_info()
    num_cores, num_subcores = info.num_cores, info.num_subcores
    num_workers = num_cores * num_subcores
    rows_per_w = EMBED_DIM // num_workers
    mesh = plsc.VectorSubcoreMesh(core_axis_name="c", subcore_axis_name="s")

    @functools.partial(
        pl.kernel,
        mesh=mesh,
        out_type=jax.ShapeDtypeStruct((EMBED_DIM, BATCH), jnp.float32),
        compiler_params=pltpu.CompilerParams(
            needs_layout_passes=False,
            disable_bounds_checks=True,
            disable_semaphore_checks=True,
            skip_device_barrier=True,
        ),
        scratch_types=[
            pltpu.VMEM((BATCH,), jnp.int32),
            pltpu.VMEM((NUM_NODES,), jnp.float32),
            pltpu.VMEM((BATCH,), jnp.float32),
            pltpu.VMEM((BATCH,), jnp.float32),
            pltpu.SemaphoreType.DMA,
            pltpu.SemaphoreType.DMA,
        ],
    )
    def gather_kernel(idx_hbm, tab_t_hbm, out_t_hbm, idx_v, row_v,
                      out0_v, out1_v, sem_r, sem_w):
        wid = lax.axis_index("s") * num_cores + lax.axis_index("c")
        j0 = wid * rows_per_w
        j1 = j0 + 1

        s0 = pltpu.async_copy(tab_t_hbm.at[j0], row_v, sem_r)
        pltpu.sync_copy(idx_hbm, idx_v)

        def gather_row(out_v):
            @plsc.parallel_loop(0, BATCH, step=LANES, unroll=8)
            def _(i):
                idxv = idx_v[pl.ds(i, LANES)]
                out_v[pl.ds(i, LANES)] = plsc.load_gather(row_v, [idxv])

        s0.wait()
        gather_row(out0_v)
        s1 = pltpu.async_copy(tab_t_hbm.at[j1], row_v, sem_r)
        w0 = pltpu.async_copy(out0_v, out_t_hbm.at[j0], sem_w)
        s1.wait()
        gather_row(out1_v)
        w0.wait()
        pltpu.sync_copy(out1_v, out_t_hbm.at[j1])

    return gather_kernel


_gather = _build()


def kernel(node_id, table):
    out_t = _gather(node_id.astype(jnp.int32), table.T)
    return out_t.T
